# Initial kernel scaffold; baseline (speedup 1.0000x reference)
import functools

import jax
import jax.numpy as jnp
from jax import lax
from jax.experimental import pallas as pl
from jax.experimental.pallas import tpu as pltpu
from jax.experimental.pallas import tpu_sc as plsc

N = 10000
E = 320000
D = 128
H = 4
C = 32

NC = 2
NS = 16
NW = NC * NS

CHUNK = 128
NCHUNKS = E // CHUNK
ROWS_PER_TILE = N // NS

_HI = lax.Precision.HIGHEST
_f32 = jnp.float32


def _k1_body(x_ref, wl_ref, wr_ref, xl_ref, xr_ref):
    xv = x_ref[...]
    xl_ref[...] = jnp.dot(xv, wl_ref[...], preferred_element_type=_f32,
                          precision=_HI)
    xr_ref[...] = jnp.dot(xv, wr_ref[...], preferred_element_type=_f32,
                          precision=_HI)


def _k1(x, W_l, W_r):
    return pl.pallas_call(
        _k1_body,
        out_shape=[jax.ShapeDtypeStruct((N, D), _f32),
                   jax.ShapeDtypeStruct((N, D), _f32)],
    )(x, W_l, W_r)


def _sc_gather2(tab_a, tab_b, idx_a, idx_b):
    mesh = plsc.VectorSubcoreMesh(core_axis_name="c", subcore_axis_name="s")

    @functools.partial(
        pl.kernel, mesh=mesh,
        out_type=[jax.ShapeDtypeStruct((E, D), _f32),
                  jax.ShapeDtypeStruct((E, D), _f32)],
        scratch_types=[pltpu.VMEM((CHUNK,), jnp.int32),
                       pltpu.VMEM((CHUNK,), jnp.int32),
                       pltpu.VMEM((CHUNK, D), _f32),
                       pltpu.VMEM((CHUNK, D), _f32),
                       pltpu.SemaphoreType.DMA,
                       pltpu.SemaphoreType.DMA],
    )
    def k(ta, tb, ia, ib, oa, ob, ia_v, ib_v, ra_v, rb_v, sa, sb):
        wid = lax.axis_index("s") * NC + lax.axis_index("c")

        def body(j, carry):
            cid = j * NW + wid

            @pl.when(cid < NCHUNKS)
            def _():
                base = cid * CHUNK
                pltpu.sync_copy(ia.at[pl.ds(base, CHUNK)], ia_v)
                pltpu.sync_copy(ib.at[pl.ds(base, CHUNK)], ib_v)
                ca = pltpu.async_copy(ta.at[ia_v], ra_v, sa)
                cb = pltpu.async_copy(tb.at[ib_v], rb_v, sb)
                ca.wait()
                cb.wait()
                pltpu.sync_copy(ra_v, oa.at[pl.ds(base, CHUNK)])
                pltpu.sync_copy(rb_v, ob.at[pl.ds(base, CHUNK)])

            return carry

        lax.fori_loop(0, (NCHUNKS + NW - 1) // NW, body, 0)

    return k(tab_a, tab_b, idx_a, idx_b)


_BE3 = 4000


def _k3_body(gl_ref, gr_ref, ea_ref, we_ref, attf_ref, m8_ref, b8_ref,
             msg_ref, ex8_ref):
    gl = gl_ref[...]
    ef = jnp.dot(ea_ref[...], we_ref[...], preferred_element_type=_f32,
                 precision=_HI)
    s = gl + gr_ref[...] + ef
    s = jnp.where(s >= 0, s, 0.2 * s)
    ew = s * attf_ref[...]
    logits8 = jnp.dot(ew, m8_ref[...], preferred_element_type=_f32,
                      precision=_HI)
    ex = jnp.exp(logits8)
    ex8_ref[...] = ex
    exb = jnp.dot(ex, b8_ref[...], preferred_element_type=_f32,
                  precision=_HI)
    msg_ref[...] = gl * exb


def _k3(g_l, g_r, edge_attr, W_e, att_flat, m8, b8):
    n_blk = E // _BE3
    return pl.pallas_call(
        _k3_body,
        grid=(n_blk,),
        in_specs=[
            pl.BlockSpec((_BE3, D), lambda i: (i, 0)),
            pl.BlockSpec((_BE3, D), lambda i: (i, 0)),
            pl.BlockSpec((_BE3, D), lambda i: (i, 0)),
            pl.BlockSpec((D, D), lambda i: (0, 0)),
            pl.BlockSpec((1, D), lambda i: (0, 0)),
            pl.BlockSpec((D, 8), lambda i: (0, 0)),
            pl.BlockSpec((8, D), lambda i: (0, 0)),
        ],
        out_specs=[
            pl.BlockSpec((_BE3, D), lambda i: (i, 0)),
            pl.BlockSpec((_BE3, 8), lambda i: (i, 0)),
        ],
        out_shape=[jax.ShapeDtypeStruct((E, D), _f32),
                   jax.ShapeDtypeStruct((E, 8), _f32)],
    )(g_l, g_r, edge_attr, W_e, att_flat, m8, b8)


def _sc_scatter(msg, ex8, dst, zS, zd):
    mesh = plsc.VectorSubcoreMesh(core_axis_name="c", subcore_axis_name="s")
    half = NCHUNKS // NC

    @functools.partial(
        pl.kernel, mesh=mesh,
        out_type=[jax.ShapeDtypeStruct((NC, N, D), _f32),
                  jax.ShapeDtypeStruct((NC, N, 8), _f32)],
        scratch_types=[pltpu.VMEM((CHUNK,), jnp.int32),
                       pltpu.VMEM((CHUNK, D), _f32),
                       pltpu.VMEM((CHUNK, 8), _f32),
                       pltpu.VMEM_SHARED((N, D), _f32),
                       pltpu.VMEM_SHARED((N, 8), _f32)],
    )
    def k(msg_h, ex_h, dst_h, zS_h, zd_h, So, Do, idx_v, m_v, e_v, accS, accD):
        c = lax.axis_index("c")
        s = lax.axis_index("s")
        r0 = s * ROWS_PER_TILE
        pltpu.sync_copy(zS_h.at[pl.ds(r0, ROWS_PER_TILE)],
                        accS.at[pl.ds(r0, ROWS_PER_TILE)])
        pltpu.sync_copy(zd_h.at[pl.ds(r0, ROWS_PER_TILE)],
                        accD.at[pl.ds(r0, ROWS_PER_TILE)])
        plsc.subcore_barrier()

        def body(j, carry):
            k_id = j * NS + s

            @pl.when(k_id < half)
            def _():
                base = (c * half + k_id) * CHUNK
                pltpu.sync_copy(dst_h.at[pl.ds(base, CHUNK)], idx_v)
                pltpu.sync_copy(msg_h.at[pl.ds(base, CHUNK)], m_v)
                pltpu.sync_copy(ex_h.at[pl.ds(base, CHUNK)], e_v)
                pltpu.sync_copy(m_v, accS.at[idx_v], add=True)
                pltpu.sync_copy(e_v, accD.at[idx_v], add=True)

            return carry

        lax.fori_loop(0, (half + NS - 1) // NS, body, 0)
        plsc.subcore_barrier()
        pltpu.sync_copy(accS.at[pl.ds(r0, ROWS_PER_TILE)],
                        So.at[c, pl.ds(r0, ROWS_PER_TILE)])
        pltpu.sync_copy(accD.at[pl.ds(r0, ROWS_PER_TILE)],
                        Do.at[c, pl.ds(r0, ROWS_PER_TILE)])

    return k(msg, ex8, dst, zS, zd)


_BN5 = 2000


def _k5_body(sp_ref, dp_ref, bias_ref, b8_ref, xout_ref):
    ssum = sp_ref[0] + sp_ref[1]
    den = dp_ref[0] + dp_ref[1]
    den_rep = jnp.dot(den, b8_ref[...], preferred_element_type=_f32,
                      precision=_HI)
    xout_ref[...] = ssum / (den_rep + 1e-16) + bias_ref[...]


def _k5(S_p, den_p, bias_row, b8):
    n_blk = N // _BN5
    return pl.pallas_call(
        _k5_body,
        grid=(n_blk,),
        in_specs=[
            pl.BlockSpec((NC, _BN5, D), lambda i: (0, i, 0)),
            pl.BlockSpec((NC, _BN5, 8), lambda i: (0, i, 0)),
            pl.BlockSpec((1, D), lambda i: (0, 0)),
            pl.BlockSpec((8, D), lambda i: (0, 0)),
        ],
        out_specs=pl.BlockSpec((_BN5, D), lambda i: (i, 0)),
        out_shape=jax.ShapeDtypeStruct((N, D), _f32),
    )(S_p, den_p, bias_row, b8)


_BE7 = 4000


def _k7_body(q_ref, k_ref, ea_ref, gq_ref, gk_ref, bq_ref, bk_ref,
             wt_ref, wb_ref, bm_ref, out_ref):
    qv = q_ref[...]
    kv = k_ref[...]
    mu = (jnp.sum(qv, axis=1, keepdims=True)
          + jnp.sum(kv, axis=1, keepdims=True)) * (1.0 / (2 * D))
    ssq = (jnp.sum(qv * qv, axis=1, keepdims=True)
           + jnp.sum(kv * kv, axis=1, keepdims=True))
    var = ssq * (1.0 / (2 * D)) - mu * mu
    inv = 1.0 / jnp.sqrt(var + 1e-5)
    qn = jnp.maximum((qv - mu) * inv * gq_ref[...] + bq_ref[...], 0.0)
    kn = jnp.maximum((kv - mu) * inv * gk_ref[...] + bk_ref[...], 0.0)
    out_ref[...] = (ea_ref[...] + bm_ref[...]
                    + jnp.dot(qn, wt_ref[...], preferred_element_type=_f32,
                              precision=_HI)
                    + jnp.dot(kn, wb_ref[...], preferred_element_type=_f32,
                              precision=_HI))


def _k7(q, k, edge_attr, gq, gk, bq, bk, W_top, W_bot, bm_row):
    n_blk = E // _BE7
    row = lambda i: (0, 0)
    return pl.pallas_call(
        _k7_body,
        grid=(n_blk,),
        in_specs=[
            pl.BlockSpec((_BE7, D), lambda i: (i, 0)),
            pl.BlockSpec((_BE7, D), lambda i: (i, 0)),
            pl.BlockSpec((_BE7, D), lambda i: (i, 0)),
            pl.BlockSpec((1, D), row),
            pl.BlockSpec((1, D), row),
            pl.BlockSpec((1, D), row),
            pl.BlockSpec((1, D), row),
            pl.BlockSpec((D, D), row),
            pl.BlockSpec((D, D), row),
            pl.BlockSpec((1, D), row),
        ],
        out_specs=pl.BlockSpec((_BE7, D), lambda i: (i, 0)),
        out_shape=jax.ShapeDtypeStruct((E, D), _f32),
    )(q, k, edge_attr, gq, gk, bq, bk, W_top, W_bot, bm_row)


def kernel(x, edge_index, edge_attr, W_l, W_r, W_e, att, bias,
           ln_gamma, ln_beta, W_mlp, b_mlp):
    src = edge_index[0]
    dst = edge_index[1]

    att_flat = att.reshape(1, H * C)
    m8 = jnp.concatenate(
        [jnp.repeat(jnp.eye(H, dtype=_f32), C, axis=0),
         jnp.zeros((D, 8 - H), _f32)], axis=1)
    b8 = m8.T
    bias_row = bias.reshape(1, D)
    gq = ln_gamma[:D].reshape(1, D)
    gk = ln_gamma[D:].reshape(1, D)
    bq = ln_beta[:D].reshape(1, D)
    bk = ln_beta[D:].reshape(1, D)
    W_top = W_mlp[:D]
    W_bot = W_mlp[D:]
    bm_row = b_mlp.reshape(1, D)
    zS = jnp.zeros((N, D), _f32)
    zd = jnp.zeros((N, 8), _f32)

    x_l, x_r = _k1(x, W_l, W_r)
    g_l, g_r = _sc_gather2(x_l, x_r, src, dst)
    msg, ex8 = _k3(g_l, g_r, edge_attr, W_e, att_flat, m8, b8)
    S_p, den_p = _sc_scatter(msg, ex8, dst, zS, zd)
    x_out = _k5(S_p, den_p, bias_row, b8)
    q, kk = _sc_gather2(x_out, x_out, src, dst)
    edge_attr_new = _k7(q, kk, edge_attr, gq, gk, bq, bk, W_top, W_bot, bm_row)
    return x_out, edge_attr_new


# speedup vs baseline: 18.4407x; 18.4407x over previous
import functools

import jax
import jax.numpy as jnp
from jax import lax
from jax.experimental import pallas as pl
from jax.experimental.pallas import tpu as pltpu
from jax.experimental.pallas import tpu_sc as plsc

N = 10000
E = 320000
D = 128
H = 4
C = 32

NC = 2
NS = 16
NW = NC * NS

CHUNK = 128
NCHUNKS = E // CHUNK
N_PAD = 10240
ROWS_PER_TILE = N_PAD // NS

_HI = lax.Precision.HIGHEST
_f32 = jnp.float32


def _k1_body(x_ref, wl_ref, wr_ref, xl_ref, xr_ref):
    xv = x_ref[...]
    xl_ref[...] = jnp.dot(xv, wl_ref[...], preferred_element_type=_f32,
                          precision=_HI)
    xr_ref[...] = jnp.dot(xv, wr_ref[...], preferred_element_type=_f32,
                          precision=_HI)


def _k1(x, W_l, W_r):
    return pl.pallas_call(
        _k1_body,
        out_shape=[jax.ShapeDtypeStruct((N, D), _f32),
                   jax.ShapeDtypeStruct((N, D), _f32)],
    )(x, W_l, W_r)


def _sc_gather2(tab_a, tab_b, idx_a, idx_b):
    mesh = plsc.VectorSubcoreMesh(core_axis_name="c", subcore_axis_name="s")

    @functools.partial(
        pl.kernel, mesh=mesh,
        out_type=[jax.ShapeDtypeStruct((E, D), _f32),
                  jax.ShapeDtypeStruct((E, D), _f32)],
        scratch_types=[pltpu.VMEM((CHUNK,), jnp.int32),
                       pltpu.VMEM((CHUNK,), jnp.int32),
                       pltpu.VMEM((CHUNK, D), _f32),
                       pltpu.VMEM((CHUNK, D), _f32),
                       pltpu.SemaphoreType.DMA,
                       pltpu.SemaphoreType.DMA],
    )
    def k(ta, tb, ia, ib, oa, ob, ia_v, ib_v, ra_v, rb_v, sa, sb):
        wid = lax.axis_index("s") * NC + lax.axis_index("c")

        def body(j, carry):
            cid = j * NW + wid

            @pl.when(cid < NCHUNKS)
            def _():
                base = cid * CHUNK
                pltpu.sync_copy(ia.at[pl.ds(base, CHUNK)], ia_v)
                pltpu.sync_copy(ib.at[pl.ds(base, CHUNK)], ib_v)
                ca = pltpu.async_copy(ta.at[ia_v], ra_v, sa)
                cb = pltpu.async_copy(tb.at[ib_v], rb_v, sb)
                ca.wait()
                cb.wait()
                pltpu.sync_copy(ra_v, oa.at[pl.ds(base, CHUNK)])
                pltpu.sync_copy(rb_v, ob.at[pl.ds(base, CHUNK)])

            return carry

        lax.fori_loop(0, (NCHUNKS + NW - 1) // NW, body, 0)

    return k(tab_a, tab_b, idx_a, idx_b)


_BE3 = 4000


def _k3_body(gl_ref, gr_ref, ea_ref, we_ref, attf_ref, m8_ref, b8_ref,
             msg_ref, exb_ref):
    gl = gl_ref[...]
    ef = jnp.dot(ea_ref[...], we_ref[...], preferred_element_type=_f32,
                 precision=_HI)
    s = gl + gr_ref[...] + ef
    s = jnp.where(s >= 0, s, 0.2 * s)
    ew = s * attf_ref[...]
    logits8 = jnp.dot(ew, m8_ref[...], preferred_element_type=_f32,
                      precision=_HI)
    ex = jnp.exp(logits8)
    exb = jnp.dot(ex, b8_ref[...], preferred_element_type=_f32,
                  precision=_HI)
    exb_ref[...] = exb
    msg_ref[...] = gl * exb


def _k3(g_l, g_r, edge_attr, W_e, att_flat, m8, b8):
    n_blk = E // _BE3
    return pl.pallas_call(
        _k3_body,
        grid=(n_blk,),
        in_specs=[
            pl.BlockSpec((_BE3, D), lambda i: (i, 0)),
            pl.BlockSpec((_BE3, D), lambda i: (i, 0)),
            pl.BlockSpec((_BE3, D), lambda i: (i, 0)),
            pl.BlockSpec((D, D), lambda i: (0, 0)),
            pl.BlockSpec((1, D), lambda i: (0, 0)),
            pl.BlockSpec((D, 8), lambda i: (0, 0)),
            pl.BlockSpec((8, D), lambda i: (0, 0)),
        ],
        out_specs=[
            pl.BlockSpec((_BE3, D), lambda i: (i, 0)),
            pl.BlockSpec((_BE3, D), lambda i: (i, 0)),
        ],
        out_shape=[jax.ShapeDtypeStruct((E, D), _f32),
                   jax.ShapeDtypeStruct((E, D), _f32)],
    )(g_l, g_r, edge_attr, W_e, att_flat, m8, b8)


def _sc_scatter(msg, exb, dst, zS):
    mesh = plsc.VectorSubcoreMesh(core_axis_name="c", subcore_axis_name="s")

    @functools.partial(
        pl.kernel, mesh=mesh,
        out_type=jax.ShapeDtypeStruct((NC, N_PAD, D), _f32),
        scratch_types=[pltpu.VMEM((CHUNK,), jnp.int32),
                       pltpu.VMEM((CHUNK, D), _f32),
                       pltpu.VMEM_SHARED((N_PAD, D), _f32)],
    )
    def k(msg_h, exb_h, dst_h, zS_h, So, idx_v, m_v, acc):
        c = lax.axis_index("c")
        s = lax.axis_index("s")
        r0 = s * ROWS_PER_TILE
        pltpu.sync_copy(zS_h, m_v)
        for t in range(ROWS_PER_TILE // CHUNK):
            pltpu.sync_copy(m_v, acc.at[pl.ds(r0 + t * CHUNK, CHUNK)])
        plsc.subcore_barrier()

        def body(j, carry):
            k_id = j * NS + s

            @pl.when(k_id < NCHUNKS)
            def _():
                base = k_id * CHUNK
                pltpu.sync_copy(dst_h.at[pl.ds(base, CHUNK)], idx_v)

                @pl.when(c == 0)
                def _():
                    pltpu.sync_copy(msg_h.at[pl.ds(base, CHUNK)], m_v)

                @pl.when(c == 1)
                def _():
                    pltpu.sync_copy(exb_h.at[pl.ds(base, CHUNK)], m_v)

                pltpu.sync_copy(m_v, acc.at[idx_v], add=True)

            return carry

        lax.fori_loop(0, (NCHUNKS + NS - 1) // NS, body, 0)
        plsc.subcore_barrier()
        for t in range(ROWS_PER_TILE // CHUNK):
            rb = r0 + t * CHUNK
            pltpu.sync_copy(acc.at[pl.ds(rb, CHUNK)], m_v)
            pltpu.sync_copy(m_v, So.at[c, pl.ds(rb, CHUNK)])

    return k(msg, exb, dst, zS)


_BN5 = 2000


def _k5_body(sp_ref, bias_ref, xout_ref):
    xout_ref[...] = sp_ref[0] / (sp_ref[1] + 1e-16) + bias_ref[...]


def _k5(S_p, bias_row):
    n_blk = N // _BN5
    return pl.pallas_call(
        _k5_body,
        grid=(n_blk,),
        in_specs=[
            pl.BlockSpec((NC, _BN5, D), lambda i: (0, i, 0)),
            pl.BlockSpec((1, D), lambda i: (0, 0)),
        ],
        out_specs=pl.BlockSpec((_BN5, D), lambda i: (i, 0)),
        out_shape=jax.ShapeDtypeStruct((N, D), _f32),
    )(S_p, bias_row)


_BE7 = 4000


def _k7_body(q_ref, k_ref, ea_ref, gq_ref, gk_ref, bq_ref, bk_ref,
             wt_ref, wb_ref, bm_ref, out_ref):
    qv = q_ref[...]
    kv = k_ref[...]
    mu = (jnp.sum(qv, axis=1, keepdims=True)
          + jnp.sum(kv, axis=1, keepdims=True)) * (1.0 / (2 * D))
    ssq = (jnp.sum(qv * qv, axis=1, keepdims=True)
           + jnp.sum(kv * kv, axis=1, keepdims=True))
    var = ssq * (1.0 / (2 * D)) - mu * mu
    inv = 1.0 / jnp.sqrt(var + 1e-5)
    qn = jnp.maximum((qv - mu) * inv * gq_ref[...] + bq_ref[...], 0.0)
    kn = jnp.maximum((kv - mu) * inv * gk_ref[...] + bk_ref[...], 0.0)
    out_ref[...] = (ea_ref[...] + bm_ref[...]
                    + jnp.dot(qn, wt_ref[...], preferred_element_type=_f32,
                              precision=_HI)
                    + jnp.dot(kn, wb_ref[...], preferred_element_type=_f32,
                              precision=_HI))


def _k7(q, k, edge_attr, gq, gk, bq, bk, W_top, W_bot, bm_row):
    n_blk = E // _BE7
    row = lambda i: (0, 0)
    return pl.pallas_call(
        _k7_body,
        grid=(n_blk,),
        in_specs=[
            pl.BlockSpec((_BE7, D), lambda i: (i, 0)),
            pl.BlockSpec((_BE7, D), lambda i: (i, 0)),
            pl.BlockSpec((_BE7, D), lambda i: (i, 0)),
            pl.BlockSpec((1, D), row),
            pl.BlockSpec((1, D), row),
            pl.BlockSpec((1, D), row),
            pl.BlockSpec((1, D), row),
            pl.BlockSpec((D, D), row),
            pl.BlockSpec((D, D), row),
            pl.BlockSpec((1, D), row),
        ],
        out_specs=pl.BlockSpec((_BE7, D), lambda i: (i, 0)),
        out_shape=jax.ShapeDtypeStruct((E, D), _f32),
    )(q, k, edge_attr, gq, gk, bq, bk, W_top, W_bot, bm_row)


def kernel(x, edge_index, edge_attr, W_l, W_r, W_e, att, bias,
           ln_gamma, ln_beta, W_mlp, b_mlp):
    src = edge_index[0]
    dst = edge_index[1]

    att_flat = att.reshape(1, H * C)
    m8 = jnp.concatenate(
        [jnp.repeat(jnp.eye(H, dtype=_f32), C, axis=0),
         jnp.zeros((D, 8 - H), _f32)], axis=1)
    b8 = m8.T
    bias_row = bias.reshape(1, D)
    gq = ln_gamma[:D].reshape(1, D)
    gk = ln_gamma[D:].reshape(1, D)
    bq = ln_beta[:D].reshape(1, D)
    bk = ln_beta[D:].reshape(1, D)
    W_top = W_mlp[:D]
    W_bot = W_mlp[D:]
    bm_row = b_mlp.reshape(1, D)
    zS = jnp.zeros((CHUNK, D), _f32)

    x_l, x_r = _k1(x, W_l, W_r)
    g_l, g_r = _sc_gather2(x_l, x_r, src, dst)
    msg, exb = _k3(g_l, g_r, edge_attr, W_e, att_flat, m8, b8)
    S_p = _sc_scatter(msg, exb, dst, zS)
    x_out = _k5(S_p, bias_row)
    q, kk = _sc_gather2(x_out, x_out, src, dst)
    edge_attr_new = _k7(q, kk, edge_attr, gq, gk, bq, bk, W_top, W_bot, bm_row)
    return x_out, edge_attr_new


# speedup vs baseline: 19.4275x; 1.0535x over previous
import functools

import jax
import jax.numpy as jnp
from jax import lax
from jax.experimental import pallas as pl
from jax.experimental.pallas import tpu as pltpu
from jax.experimental.pallas import tpu_sc as plsc

N = 10000
E = 320000
D = 128
H = 4
C = 32

NC = 2
NS = 16
NW = NC * NS

CHUNK = 128
NCHUNKS = E // CHUNK
N_PAD = 10240
ROWS_PER_TILE = N_PAD // NS

_HI = lax.Precision.HIGHEST
_f32 = jnp.float32


def _k1_body(x_ref, wl_ref, wr_ref, xl_ref, xr_ref):
    xv = x_ref[...]
    xl_ref[...] = jnp.dot(xv, wl_ref[...], preferred_element_type=_f32,
                          precision=_HI)
    xr_ref[...] = jnp.dot(xv, wr_ref[...], preferred_element_type=_f32,
                          precision=_HI)


def _k1(x, W_l, W_r):
    return pl.pallas_call(
        _k1_body,
        out_shape=[jax.ShapeDtypeStruct((N, D), _f32),
                   jax.ShapeDtypeStruct((N, D), _f32)],
    )(x, W_l, W_r)


SUP = 256
NSUP = E // SUP


def _sc_gather2(tab_a, tab_b, idx3_a, idx3_b):
    mesh = plsc.VectorSubcoreMesh(core_axis_name="c", subcore_axis_name="s")
    nsub = SUP // CHUNK

    @functools.partial(
        pl.kernel, mesh=mesh,
        out_type=[jax.ShapeDtypeStruct((E, D), _f32),
                  jax.ShapeDtypeStruct((E, D), _f32)],
        scratch_types=[pltpu.VMEM((nsub, CHUNK), jnp.int32),
                       pltpu.VMEM((nsub, CHUNK), jnp.int32),
                       pltpu.VMEM((SUP, D), _f32),
                       pltpu.VMEM((SUP, D), _f32),
                       pltpu.SemaphoreType.DMA,
                       pltpu.SemaphoreType.DMA],
    )
    def k(ta, tb, ia, ib, oa, ob, ia_v, ib_v, ra_v, rb_v, sa, sb):
        wid = lax.axis_index("s") * NC + lax.axis_index("c")

        def body(j, carry):
            sid = j * NW + wid

            @pl.when(sid < NSUP)
            def _():
                base = sid * SUP
                pltpu.sync_copy(ia.at[sid], ia_v)
                pltpu.sync_copy(ib.at[sid], ib_v)
                cps = []
                for t in range(nsub):
                    sl = pl.ds(t * CHUNK, CHUNK)
                    cps.append(pltpu.async_copy(
                        ta.at[ia_v.at[t]], ra_v.at[sl], sa))
                    cps.append(pltpu.async_copy(
                        tb.at[ib_v.at[t]], rb_v.at[sl], sb))
                for cp in cps:
                    cp.wait()
                pltpu.sync_copy(ra_v, oa.at[pl.ds(base, SUP)])
                pltpu.sync_copy(rb_v, ob.at[pl.ds(base, SUP)])

            return carry

        lax.fori_loop(0, (NSUP + NW - 1) // NW, body, 0)

    return k(tab_a, tab_b, idx3_a, idx3_b)


_BE3 = 4000


def _k3_body(gl_ref, gr_ref, ea_ref, we_ref, attf_ref, m8_ref, b8_ref,
             msg_ref, exb_ref):
    gl = gl_ref[...]
    ef = jnp.dot(ea_ref[...], we_ref[...], preferred_element_type=_f32,
                 precision=_HI)
    s = gl + gr_ref[...] + ef
    s = jnp.where(s >= 0, s, 0.2 * s)
    ew = s * attf_ref[...]
    logits8 = jnp.dot(ew, m8_ref[...], preferred_element_type=_f32,
                      precision=_HI)
    ex = jnp.exp(logits8)
    exb = jnp.dot(ex, b8_ref[...], preferred_element_type=_f32,
                  precision=_HI)
    exb_ref[...] = exb
    msg_ref[...] = gl * exb


def _k3(g_l, g_r, edge_attr, W_e, att_flat, m8, b8):
    n_blk = E // _BE3
    return pl.pallas_call(
        _k3_body,
        grid=(n_blk,),
        in_specs=[
            pl.BlockSpec((_BE3, D), lambda i: (i, 0)),
            pl.BlockSpec((_BE3, D), lambda i: (i, 0)),
            pl.BlockSpec((_BE3, D), lambda i: (i, 0)),
            pl.BlockSpec((D, D), lambda i: (0, 0)),
            pl.BlockSpec((1, D), lambda i: (0, 0)),
            pl.BlockSpec((D, 8), lambda i: (0, 0)),
            pl.BlockSpec((8, D), lambda i: (0, 0)),
        ],
        out_specs=[
            pl.BlockSpec((_BE3, D), lambda i: (i, 0)),
            pl.BlockSpec((_BE3, D), lambda i: (i, 0)),
        ],
        out_shape=[jax.ShapeDtypeStruct((E, D), _f32),
                   jax.ShapeDtypeStruct((E, D), _f32)],
    )(g_l, g_r, edge_attr, W_e, att_flat, m8, b8)


def _sc_scatter(msg, exb, dst, zS):
    mesh = plsc.VectorSubcoreMesh(core_axis_name="c", subcore_axis_name="s")

    @functools.partial(
        pl.kernel, mesh=mesh,
        out_type=jax.ShapeDtypeStruct((NC, N_PAD, D), _f32),
        scratch_types=[pltpu.VMEM((CHUNK,), jnp.int32),
                       pltpu.VMEM((CHUNK, D), _f32),
                       pltpu.VMEM_SHARED((N_PAD, D), _f32)],
    )
    def k(msg_h, exb_h, dst_h, zS_h, So, idx_v, m_v, acc):
        c = lax.axis_index("c")
        s = lax.axis_index("s")
        r0 = s * ROWS_PER_TILE
        pltpu.sync_copy(zS_h, m_v)
        for t in range(ROWS_PER_TILE // CHUNK):
            pltpu.sync_copy(m_v, acc.at[pl.ds(r0 + t * CHUNK, CHUNK)])
        plsc.subcore_barrier()

        def body(j, carry):
            k_id = j * NS + s

            @pl.when(k_id < NCHUNKS)
            def _():
                base = k_id * CHUNK
                pltpu.sync_copy(dst_h.at[pl.ds(base, CHUNK)], idx_v)

                @pl.when(c == 0)
                def _():
                    pltpu.sync_copy(msg_h.at[pl.ds(base, CHUNK)], m_v)

                @pl.when(c == 1)
                def _():
                    pltpu.sync_copy(exb_h.at[pl.ds(base, CHUNK)], m_v)

                pltpu.sync_copy(m_v, acc.at[idx_v], add=True)

            return carry

        lax.fori_loop(0, (NCHUNKS + NS - 1) // NS, body, 0)
        plsc.subcore_barrier()
        for t in range(ROWS_PER_TILE // CHUNK):
            rb = r0 + t * CHUNK
            pltpu.sync_copy(acc.at[pl.ds(rb, CHUNK)], m_v)
            pltpu.sync_copy(m_v, So.at[c, pl.ds(rb, CHUNK)])

    return k(msg, exb, dst, zS)


_BN5 = 2000


def _k5_body(sp_ref, bias_ref, xout_ref):
    xout_ref[...] = sp_ref[0] / (sp_ref[1] + 1e-16) + bias_ref[...]


def _k5(S_p, bias_row):
    n_blk = N // _BN5
    return pl.pallas_call(
        _k5_body,
        grid=(n_blk,),
        in_specs=[
            pl.BlockSpec((NC, _BN5, D), lambda i: (0, i, 0)),
            pl.BlockSpec((1, D), lambda i: (0, 0)),
        ],
        out_specs=pl.BlockSpec((_BN5, D), lambda i: (i, 0)),
        out_shape=jax.ShapeDtypeStruct((N, D), _f32),
    )(S_p, bias_row)


_BE7 = 4000


def _k7_body(q_ref, k_ref, ea_ref, gq_ref, gk_ref, bq_ref, bk_ref,
             wt_ref, wb_ref, bm_ref, out_ref):
    qv = q_ref[...]
    kv = k_ref[...]
    mu = (jnp.sum(qv, axis=1, keepdims=True)
          + jnp.sum(kv, axis=1, keepdims=True)) * (1.0 / (2 * D))
    ssq = (jnp.sum(qv * qv, axis=1, keepdims=True)
           + jnp.sum(kv * kv, axis=1, keepdims=True))
    var = ssq * (1.0 / (2 * D)) - mu * mu
    inv = 1.0 / jnp.sqrt(var + 1e-5)
    qn = jnp.maximum((qv - mu) * inv * gq_ref[...] + bq_ref[...], 0.0)
    kn = jnp.maximum((kv - mu) * inv * gk_ref[...] + bk_ref[...], 0.0)
    out_ref[...] = (ea_ref[...] + bm_ref[...]
                    + jnp.dot(qn, wt_ref[...], preferred_element_type=_f32,
                              precision=_HI)
                    + jnp.dot(kn, wb_ref[...], preferred_element_type=_f32,
                              precision=_HI))


def _k7(q, k, edge_attr, gq, gk, bq, bk, W_top, W_bot, bm_row):
    n_blk = E // _BE7
    row = lambda i: (0, 0)
    return pl.pallas_call(
        _k7_body,
        grid=(n_blk,),
        in_specs=[
            pl.BlockSpec((_BE7, D), lambda i: (i, 0)),
            pl.BlockSpec((_BE7, D), lambda i: (i, 0)),
            pl.BlockSpec((_BE7, D), lambda i: (i, 0)),
            pl.BlockSpec((1, D), row),
            pl.BlockSpec((1, D), row),
            pl.BlockSpec((1, D), row),
            pl.BlockSpec((1, D), row),
            pl.BlockSpec((D, D), row),
            pl.BlockSpec((D, D), row),
            pl.BlockSpec((1, D), row),
        ],
        out_specs=pl.BlockSpec((_BE7, D), lambda i: (i, 0)),
        out_shape=jax.ShapeDtypeStruct((E, D), _f32),
    )(q, k, edge_attr, gq, gk, bq, bk, W_top, W_bot, bm_row)


def kernel(x, edge_index, edge_attr, W_l, W_r, W_e, att, bias,
           ln_gamma, ln_beta, W_mlp, b_mlp):
    src = edge_index[0]
    dst = edge_index[1]

    att_flat = att.reshape(1, H * C)
    m8 = jnp.concatenate(
        [jnp.repeat(jnp.eye(H, dtype=_f32), C, axis=0),
         jnp.zeros((D, 8 - H), _f32)], axis=1)
    b8 = m8.T
    bias_row = bias.reshape(1, D)
    gq = ln_gamma[:D].reshape(1, D)
    gk = ln_gamma[D:].reshape(1, D)
    bq = ln_beta[:D].reshape(1, D)
    bk = ln_beta[D:].reshape(1, D)
    W_top = W_mlp[:D]
    W_bot = W_mlp[D:]
    bm_row = b_mlp.reshape(1, D)
    zS = jnp.zeros((CHUNK, D), _f32)

    src3 = src.reshape(NSUP, SUP // CHUNK, CHUNK)
    dst3 = dst.reshape(NSUP, SUP // CHUNK, CHUNK)

    x_l, x_r = _k1(x, W_l, W_r)
    g_l, g_r = _sc_gather2(x_l, x_r, src3, dst3)
    msg, exb = _k3(g_l, g_r, edge_attr, W_e, att_flat, m8, b8)
    S_p = _sc_scatter(msg, exb, dst, zS)
    x_out = _k5(S_p, bias_row)
    q, kk = _sc_gather2(x_out, x_out, src3, dst3)
    edge_attr_new = _k7(q, kk, edge_attr, gq, gk, bq, bk, W_top, W_bot, bm_row)
    return x_out, edge_attr_new


# speedup vs baseline: 19.9993x; 1.0294x over previous
import functools

import jax
import jax.numpy as jnp
from jax import lax
from jax.experimental import pallas as pl
from jax.experimental.pallas import tpu as pltpu
from jax.experimental.pallas import tpu_sc as plsc

N = 10000
E = 320000
D = 128
H = 4
C = 32

NC = 2
NS = 16
NW = NC * NS

CHUNK = 128
NCHUNKS = E // CHUNK
N_PAD = 10240
ROWS_PER_TILE = N_PAD // NS

_HI = lax.Precision.HIGHEST
_f32 = jnp.float32


def _k1_body(x_ref, wl_ref, wr_ref, xl_ref, xr_ref):
    xv = x_ref[...]
    xl_ref[...] = jnp.dot(xv, wl_ref[...], preferred_element_type=_f32,
                          precision=_HI)
    xr_ref[...] = jnp.dot(xv, wr_ref[...], preferred_element_type=_f32,
                          precision=_HI)


def _k1(x_pad, W_l, W_r):
    return pl.pallas_call(
        _k1_body,
        out_shape=[jax.ShapeDtypeStruct((N_PAD, D), _f32),
                   jax.ShapeDtypeStruct((N_PAD, D), _f32)],
    )(x_pad, W_l, W_r)


SUP = 256
NSUP = E // SUP


def _sc_gather2(tab_a, tab_b, idx3_a, idx3_b):
    mesh = plsc.VectorSubcoreMesh(core_axis_name="c", subcore_axis_name="s")
    nsub = SUP // CHUNK

    @functools.partial(
        pl.kernel, mesh=mesh,
        out_type=[jax.ShapeDtypeStruct((E, D), _f32),
                  jax.ShapeDtypeStruct((E, D), _f32)],
        scratch_types=[pltpu.VMEM((nsub, CHUNK), jnp.int32),
                       pltpu.VMEM((SUP, D), _f32),
                       pltpu.SemaphoreType.DMA,
                       pltpu.VMEM_SHARED((N_PAD, D), _f32)],
    )
    def k(ta, tb, ia, ib, oa, ob, idx_v, r_v, sem, tab_sp):
        c = lax.axis_index("c")
        s = lax.axis_index("s")
        r0 = s * ROWS_PER_TILE
        for t in range(ROWS_PER_TILE // CHUNK):
            sl_h = pl.ds(r0 + t * CHUNK, CHUNK)
            sl_v = pl.ds(0, CHUNK)

            @pl.when(c == 0)
            def _():
                pltpu.sync_copy(ta.at[sl_h], r_v.at[sl_v])

            @pl.when(c == 1)
            def _():
                pltpu.sync_copy(tb.at[sl_h], r_v.at[sl_v])

            pltpu.sync_copy(r_v.at[sl_v], tab_sp.at[sl_h])
        plsc.subcore_barrier()

        def body(j, carry):
            sid = j * NS + s

            @pl.when(sid < NSUP)
            def _():
                base = sid * SUP

                @pl.when(c == 0)
                def _():
                    pltpu.sync_copy(ia.at[sid], idx_v)

                @pl.when(c == 1)
                def _():
                    pltpu.sync_copy(ib.at[sid], idx_v)

                cps = []
                for t in range(nsub):
                    cps.append(pltpu.async_copy(
                        tab_sp.at[idx_v.at[t]],
                        r_v.at[pl.ds(t * CHUNK, CHUNK)], sem))
                for cp in cps:
                    cp.wait()

                @pl.when(c == 0)
                def _():
                    pltpu.sync_copy(r_v, oa.at[pl.ds(base, SUP)])

                @pl.when(c == 1)
                def _():
                    pltpu.sync_copy(r_v, ob.at[pl.ds(base, SUP)])

            return carry

        lax.fori_loop(0, (NSUP + NS - 1) // NS, body, 0)

    return k(tab_a, tab_b, idx3_a, idx3_b)


_BE3 = 4000


def _k3_body(gl_ref, gr_ref, ea_ref, we_ref, attf_ref, m8_ref, b8_ref,
             msg_ref, exb_ref):
    gl = gl_ref[...]
    ef = jnp.dot(ea_ref[...], we_ref[...], preferred_element_type=_f32,
                 precision=_HI)
    s = gl + gr_ref[...] + ef
    s = jnp.where(s >= 0, s, 0.2 * s)
    ew = s * attf_ref[...]
    logits8 = jnp.dot(ew, m8_ref[...], preferred_element_type=_f32,
                      precision=_HI)
    ex = jnp.exp(logits8)
    exb = jnp.dot(ex, b8_ref[...], preferred_element_type=_f32,
                  precision=_HI)
    exb_ref[...] = exb
    msg_ref[...] = gl * exb


def _k3(g_l, g_r, edge_attr, W_e, att_flat, m8, b8):
    n_blk = E // _BE3
    return pl.pallas_call(
        _k3_body,
        grid=(n_blk,),
        in_specs=[
            pl.BlockSpec((_BE3, D), lambda i: (i, 0)),
            pl.BlockSpec((_BE3, D), lambda i: (i, 0)),
            pl.BlockSpec((_BE3, D), lambda i: (i, 0)),
            pl.BlockSpec((D, D), lambda i: (0, 0)),
            pl.BlockSpec((1, D), lambda i: (0, 0)),
            pl.BlockSpec((D, 8), lambda i: (0, 0)),
            pl.BlockSpec((8, D), lambda i: (0, 0)),
        ],
        out_specs=[
            pl.BlockSpec((_BE3, D), lambda i: (i, 0)),
            pl.BlockSpec((_BE3, D), lambda i: (i, 0)),
        ],
        out_shape=[jax.ShapeDtypeStruct((E, D), _f32),
                   jax.ShapeDtypeStruct((E, D), _f32)],
    )(g_l, g_r, edge_attr, W_e, att_flat, m8, b8)


def _sc_scatter(msg, exb, dst, zS):
    mesh = plsc.VectorSubcoreMesh(core_axis_name="c", subcore_axis_name="s")

    @functools.partial(
        pl.kernel, mesh=mesh,
        out_type=jax.ShapeDtypeStruct((NC, N_PAD, D), _f32),
        scratch_types=[pltpu.VMEM((CHUNK,), jnp.int32),
                       pltpu.VMEM((CHUNK, D), _f32),
                       pltpu.VMEM_SHARED((N_PAD, D), _f32)],
    )
    def k(msg_h, exb_h, dst_h, zS_h, So, idx_v, m_v, acc):
        c = lax.axis_index("c")
        s = lax.axis_index("s")
        r0 = s * ROWS_PER_TILE
        pltpu.sync_copy(zS_h, m_v)
        for t in range(ROWS_PER_TILE // CHUNK):
            pltpu.sync_copy(m_v, acc.at[pl.ds(r0 + t * CHUNK, CHUNK)])
        plsc.subcore_barrier()

        def body(j, carry):
            k_id = j * NS + s

            @pl.when(k_id < NCHUNKS)
            def _():
                base = k_id * CHUNK
                pltpu.sync_copy(dst_h.at[pl.ds(base, CHUNK)], idx_v)

                @pl.when(c == 0)
                def _():
                    pltpu.sync_copy(msg_h.at[pl.ds(base, CHUNK)], m_v)

                @pl.when(c == 1)
                def _():
                    pltpu.sync_copy(exb_h.at[pl.ds(base, CHUNK)], m_v)

                pltpu.sync_copy(m_v, acc.at[idx_v], add=True)

            return carry

        lax.fori_loop(0, (NCHUNKS + NS - 1) // NS, body, 0)
        plsc.subcore_barrier()
        for t in range(ROWS_PER_TILE // CHUNK):
            rb = r0 + t * CHUNK
            pltpu.sync_copy(acc.at[pl.ds(rb, CHUNK)], m_v)
            pltpu.sync_copy(m_v, So.at[c, pl.ds(rb, CHUNK)])

    return k(msg, exb, dst, zS)


_BN5 = 2048


def _k5_body(sp_ref, bias_ref, xout_ref):
    xout_ref[...] = sp_ref[0] / (sp_ref[1] + 1e-16) + bias_ref[...]


def _k5(S_p, bias_row):
    n_blk = N_PAD // _BN5
    return pl.pallas_call(
        _k5_body,
        grid=(n_blk,),
        in_specs=[
            pl.BlockSpec((NC, _BN5, D), lambda i: (0, i, 0)),
            pl.BlockSpec((1, D), lambda i: (0, 0)),
        ],
        out_specs=pl.BlockSpec((_BN5, D), lambda i: (i, 0)),
        out_shape=jax.ShapeDtypeStruct((N_PAD, D), _f32),
    )(S_p, bias_row)


_BE7 = 4000


def _k7_body(q_ref, k_ref, ea_ref, gq_ref, gk_ref, bq_ref, bk_ref,
             wt_ref, wb_ref, bm_ref, out_ref):
    qv = q_ref[...]
    kv = k_ref[...]
    mu = (jnp.sum(qv, axis=1, keepdims=True)
          + jnp.sum(kv, axis=1, keepdims=True)) * (1.0 / (2 * D))
    ssq = (jnp.sum(qv * qv, axis=1, keepdims=True)
           + jnp.sum(kv * kv, axis=1, keepdims=True))
    var = ssq * (1.0 / (2 * D)) - mu * mu
    inv = 1.0 / jnp.sqrt(var + 1e-5)
    qn = jnp.maximum((qv - mu) * inv * gq_ref[...] + bq_ref[...], 0.0)
    kn = jnp.maximum((kv - mu) * inv * gk_ref[...] + bk_ref[...], 0.0)
    out_ref[...] = (ea_ref[...] + bm_ref[...]
                    + jnp.dot(qn, wt_ref[...], preferred_element_type=_f32,
                              precision=_HI)
                    + jnp.dot(kn, wb_ref[...], preferred_element_type=_f32,
                              precision=_HI))


def _k7(q, k, edge_attr, gq, gk, bq, bk, W_top, W_bot, bm_row):
    n_blk = E // _BE7
    row = lambda i: (0, 0)
    return pl.pallas_call(
        _k7_body,
        grid=(n_blk,),
        in_specs=[
            pl.BlockSpec((_BE7, D), lambda i: (i, 0)),
            pl.BlockSpec((_BE7, D), lambda i: (i, 0)),
            pl.BlockSpec((_BE7, D), lambda i: (i, 0)),
            pl.BlockSpec((1, D), row),
            pl.BlockSpec((1, D), row),
            pl.BlockSpec((1, D), row),
            pl.BlockSpec((1, D), row),
            pl.BlockSpec((D, D), row),
            pl.BlockSpec((D, D), row),
            pl.BlockSpec((1, D), row),
        ],
        out_specs=pl.BlockSpec((_BE7, D), lambda i: (i, 0)),
        out_shape=jax.ShapeDtypeStruct((E, D), _f32),
    )(q, k, edge_attr, gq, gk, bq, bk, W_top, W_bot, bm_row)


def kernel(x, edge_index, edge_attr, W_l, W_r, W_e, att, bias,
           ln_gamma, ln_beta, W_mlp, b_mlp):
    src = edge_index[0]
    dst = edge_index[1]

    att_flat = att.reshape(1, H * C)
    m8 = jnp.concatenate(
        [jnp.repeat(jnp.eye(H, dtype=_f32), C, axis=0),
         jnp.zeros((D, 8 - H), _f32)], axis=1)
    b8 = m8.T
    bias_row = bias.reshape(1, D)
    gq = ln_gamma[:D].reshape(1, D)
    gk = ln_gamma[D:].reshape(1, D)
    bq = ln_beta[:D].reshape(1, D)
    bk = ln_beta[D:].reshape(1, D)
    W_top = W_mlp[:D]
    W_bot = W_mlp[D:]
    bm_row = b_mlp.reshape(1, D)
    zS = jnp.zeros((CHUNK, D), _f32)

    src3 = src.reshape(NSUP, SUP // CHUNK, CHUNK)
    dst3 = dst.reshape(NSUP, SUP // CHUNK, CHUNK)
    x_pad = jnp.pad(x, ((0, N_PAD - N), (0, 0)))

    x_l, x_r = _k1(x_pad, W_l, W_r)
    g_l, g_r = _sc_gather2(x_l, x_r, src3, dst3)
    msg, exb = _k3(g_l, g_r, edge_attr, W_e, att_flat, m8, b8)
    S_p = _sc_scatter(msg, exb, dst, zS)
    x_out_pad = _k5(S_p, bias_row)
    q, kk = _sc_gather2(x_out_pad, x_out_pad, src3, dst3)
    edge_attr_new = _k7(q, kk, edge_attr, gq, gk, bq, bk, W_top, W_bot, bm_row)
    return x_out_pad[:N], edge_attr_new


# speedup vs baseline: 20.0585x; 1.0030x over previous
import functools

import jax
import jax.numpy as jnp
from jax import lax
from jax.experimental import pallas as pl
from jax.experimental.pallas import tpu as pltpu
from jax.experimental.pallas import tpu_sc as plsc

N = 10000
E = 320000
D = 128
H = 4
C = 32

NC = 2
NS = 16
NW = NC * NS

CHUNK = 128
NCHUNKS = E // CHUNK
N_PAD = 10240
ROWS_PER_TILE = N_PAD // NS

_HI = lax.Precision.HIGHEST
_f32 = jnp.float32


def _k1_body(x_ref, wl_ref, wr_ref, xl_ref, xr_ref):
    xv = x_ref[...]
    xl_ref[...] = jnp.dot(xv, wl_ref[...], preferred_element_type=_f32,
                          precision=_HI)
    xr_ref[...] = jnp.dot(xv, wr_ref[...], preferred_element_type=_f32,
                          precision=_HI)


def _k1(x_pad, W_l, W_r):
    return pl.pallas_call(
        _k1_body,
        out_shape=[jax.ShapeDtypeStruct((N_PAD, D), _f32),
                   jax.ShapeDtypeStruct((N_PAD, D), _f32)],
    )(x_pad, W_l, W_r)


SUP = 640
NSUP = E // SUP


def _sc_gather2(tab_a, tab_b, idx3_a, idx3_b):
    mesh = plsc.VectorSubcoreMesh(core_axis_name="c", subcore_axis_name="s")
    nsub = SUP // CHUNK

    @functools.partial(
        pl.kernel, mesh=mesh,
        out_type=[jax.ShapeDtypeStruct((E, D), _f32),
                  jax.ShapeDtypeStruct((E, D), _f32)],
        scratch_types=[pltpu.VMEM((nsub, CHUNK), jnp.int32),
                       pltpu.VMEM((SUP, D), _f32),
                       pltpu.SemaphoreType.DMA],
    )
    def k(ta, tb, ia, ib, oa, ob, idx_v, r_v, sem):
        c = lax.axis_index("c")
        s = lax.axis_index("s")

        def one_stream(tab, idx3, out, sid):
            base = sid * SUP
            pltpu.sync_copy(idx3.at[sid], idx_v)
            cps = []
            for t in range(nsub):
                cps.append(pltpu.async_copy(
                    tab.at[idx_v.at[t]],
                    r_v.at[pl.ds(t * CHUNK, CHUNK)], sem))
            for cp in cps:
                cp.wait()
            pltpu.sync_copy(r_v, out.at[pl.ds(base, SUP)])

        def body(j, carry):
            sid = j * NS + s

            @pl.when(sid < NSUP)
            def _():
                @pl.when(c == 0)
                def _():
                    one_stream(ta, ia, oa, sid)

                @pl.when(c == 1)
                def _():
                    one_stream(tb, ib, ob, sid)

            return carry

        lax.fori_loop(0, (NSUP + NS - 1) // NS, body, 0)

    return k(tab_a, tab_b, idx3_a, idx3_b)


_BE3 = 4000


def _k3_body(gl_ref, gr_ref, ea_ref, we_ref, attf_ref, m8_ref, b8_ref,
             msg_ref, exb_ref):
    gl = gl_ref[...]
    ef = jnp.dot(ea_ref[...], we_ref[...], preferred_element_type=_f32,
                 precision=_HI)
    s = gl + gr_ref[...] + ef
    s = jnp.where(s >= 0, s, 0.2 * s)
    ew = s * attf_ref[...]
    logits8 = jnp.dot(ew, m8_ref[...], preferred_element_type=_f32,
                      precision=_HI)
    ex = jnp.exp(logits8)
    exb = jnp.dot(ex, b8_ref[...], preferred_element_type=_f32,
                  precision=_HI)
    exb_ref[...] = exb
    msg_ref[...] = gl * exb


def _k3(g_l, g_r, edge_attr, W_e, att_flat, m8, b8):
    n_blk = E // _BE3
    return pl.pallas_call(
        _k3_body,
        grid=(n_blk,),
        in_specs=[
            pl.BlockSpec((_BE3, D), lambda i: (i, 0)),
            pl.BlockSpec((_BE3, D), lambda i: (i, 0)),
            pl.BlockSpec((_BE3, D), lambda i: (i, 0)),
            pl.BlockSpec((D, D), lambda i: (0, 0)),
            pl.BlockSpec((1, D), lambda i: (0, 0)),
            pl.BlockSpec((D, 8), lambda i: (0, 0)),
            pl.BlockSpec((8, D), lambda i: (0, 0)),
        ],
        out_specs=[
            pl.BlockSpec((_BE3, D), lambda i: (i, 0)),
            pl.BlockSpec((_BE3, D), lambda i: (i, 0)),
        ],
        out_shape=[jax.ShapeDtypeStruct((E, D), _f32),
                   jax.ShapeDtypeStruct((E, D), _f32)],
    )(g_l, g_r, edge_attr, W_e, att_flat, m8, b8)


def _sc_scatter(msg, exb, dst, zS):
    mesh = plsc.VectorSubcoreMesh(core_axis_name="c", subcore_axis_name="s")

    @functools.partial(
        pl.kernel, mesh=mesh,
        out_type=jax.ShapeDtypeStruct((NC, N_PAD, D), _f32),
        scratch_types=[pltpu.VMEM((CHUNK,), jnp.int32),
                       pltpu.VMEM((CHUNK, D), _f32),
                       pltpu.VMEM_SHARED((N_PAD, D), _f32)],
    )
    def k(msg_h, exb_h, dst_h, zS_h, So, idx_v, m_v, acc):
        c = lax.axis_index("c")
        s = lax.axis_index("s")
        r0 = s * ROWS_PER_TILE
        pltpu.sync_copy(zS_h, m_v)
        for t in range(ROWS_PER_TILE // CHUNK):
            pltpu.sync_copy(m_v, acc.at[pl.ds(r0 + t * CHUNK, CHUNK)])
        plsc.subcore_barrier()

        def body(j, carry):
            k_id = j * NS + s

            @pl.when(k_id < NCHUNKS)
            def _():
                base = k_id * CHUNK
                pltpu.sync_copy(dst_h.at[pl.ds(base, CHUNK)], idx_v)

                @pl.when(c == 0)
                def _():
                    pltpu.sync_copy(msg_h.at[pl.ds(base, CHUNK)], m_v)

                @pl.when(c == 1)
                def _():
                    pltpu.sync_copy(exb_h.at[pl.ds(base, CHUNK)], m_v)

                pltpu.sync_copy(m_v, acc.at[idx_v], add=True)

            return carry

        lax.fori_loop(0, (NCHUNKS + NS - 1) // NS, body, 0)
        plsc.subcore_barrier()
        for t in range(ROWS_PER_TILE // CHUNK):
            rb = r0 + t * CHUNK
            pltpu.sync_copy(acc.at[pl.ds(rb, CHUNK)], m_v)
            pltpu.sync_copy(m_v, So.at[c, pl.ds(rb, CHUNK)])

    return k(msg, exb, dst, zS)


_BN5 = 2048


def _k5_body(sp_ref, bias_ref, xout_ref):
    xout_ref[...] = sp_ref[0] / (sp_ref[1] + 1e-16) + bias_ref[...]


def _k5(S_p, bias_row):
    n_blk = N_PAD // _BN5
    return pl.pallas_call(
        _k5_body,
        grid=(n_blk,),
        in_specs=[
            pl.BlockSpec((NC, _BN5, D), lambda i: (0, i, 0)),
            pl.BlockSpec((1, D), lambda i: (0, 0)),
        ],
        out_specs=pl.BlockSpec((_BN5, D), lambda i: (i, 0)),
        out_shape=jax.ShapeDtypeStruct((N_PAD, D), _f32),
    )(S_p, bias_row)


_BE7 = 4000


def _k7_body(q_ref, k_ref, ea_ref, gq_ref, gk_ref, bq_ref, bk_ref,
             wt_ref, wb_ref, bm_ref, out_ref):
    qv = q_ref[...]
    kv = k_ref[...]
    mu = (jnp.sum(qv, axis=1, keepdims=True)
          + jnp.sum(kv, axis=1, keepdims=True)) * (1.0 / (2 * D))
    ssq = (jnp.sum(qv * qv, axis=1, keepdims=True)
           + jnp.sum(kv * kv, axis=1, keepdims=True))
    var = ssq * (1.0 / (2 * D)) - mu * mu
    inv = 1.0 / jnp.sqrt(var + 1e-5)
    qn = jnp.maximum((qv - mu) * inv * gq_ref[...] + bq_ref[...], 0.0)
    kn = jnp.maximum((kv - mu) * inv * gk_ref[...] + bk_ref[...], 0.0)
    out_ref[...] = (ea_ref[...] + bm_ref[...]
                    + jnp.dot(qn, wt_ref[...], preferred_element_type=_f32,
                              precision=_HI)
                    + jnp.dot(kn, wb_ref[...], preferred_element_type=_f32,
                              precision=_HI))


def _k7(q, k, edge_attr, gq, gk, bq, bk, W_top, W_bot, bm_row):
    n_blk = E // _BE7
    row = lambda i: (0, 0)
    return pl.pallas_call(
        _k7_body,
        grid=(n_blk,),
        in_specs=[
            pl.BlockSpec((_BE7, D), lambda i: (i, 0)),
            pl.BlockSpec((_BE7, D), lambda i: (i, 0)),
            pl.BlockSpec((_BE7, D), lambda i: (i, 0)),
            pl.BlockSpec((1, D), row),
            pl.BlockSpec((1, D), row),
            pl.BlockSpec((1, D), row),
            pl.BlockSpec((1, D), row),
            pl.BlockSpec((D, D), row),
            pl.BlockSpec((D, D), row),
            pl.BlockSpec((1, D), row),
        ],
        out_specs=pl.BlockSpec((_BE7, D), lambda i: (i, 0)),
        out_shape=jax.ShapeDtypeStruct((E, D), _f32),
    )(q, k, edge_attr, gq, gk, bq, bk, W_top, W_bot, bm_row)


def kernel(x, edge_index, edge_attr, W_l, W_r, W_e, att, bias,
           ln_gamma, ln_beta, W_mlp, b_mlp):
    src = edge_index[0]
    dst = edge_index[1]

    att_flat = att.reshape(1, H * C)
    m8 = jnp.concatenate(
        [jnp.repeat(jnp.eye(H, dtype=_f32), C, axis=0),
         jnp.zeros((D, 8 - H), _f32)], axis=1)
    b8 = m8.T
    bias_row = bias.reshape(1, D)
    gq = ln_gamma[:D].reshape(1, D)
    gk = ln_gamma[D:].reshape(1, D)
    bq = ln_beta[:D].reshape(1, D)
    bk = ln_beta[D:].reshape(1, D)
    W_top = W_mlp[:D]
    W_bot = W_mlp[D:]
    bm_row = b_mlp.reshape(1, D)
    zS = jnp.zeros((CHUNK, D), _f32)

    src3 = src.reshape(NSUP, SUP // CHUNK, CHUNK)
    dst3 = dst.reshape(NSUP, SUP // CHUNK, CHUNK)
    x_pad = jnp.pad(x, ((0, N_PAD - N), (0, 0)))

    x_l, x_r = _k1(x_pad, W_l, W_r)
    g_l, g_r = _sc_gather2(x_l, x_r, src3, dst3)
    msg, exb = _k3(g_l, g_r, edge_attr, W_e, att_flat, m8, b8)
    S_p = _sc_scatter(msg, exb, dst, zS)
    x_out_pad = _k5(S_p, bias_row)
    q, kk = _sc_gather2(x_out_pad, x_out_pad, src3, dst3)
    edge_attr_new = _k7(q, kk, edge_attr, gq, gk, bq, bk, W_top, W_bot, bm_row)
    return x_out_pad[:N], edge_attr_new
